# baseline (speedup 1.0000x reference)
import functools

import jax
import jax.numpy as jnp
from jax import lax
from jax.experimental import pallas as pl
from jax.experimental.pallas import tpu as pltpu
from jax.experimental.pallas import tpu_sc as plsc

MAXLEN = 200
VOCAB = 100000
D = 64
BATCH = 4096

NC = 2
NS = 16
NW = NC * NS
ROWS_PER_W = BATCH // NW
PHASES = 2
HALF = MAXLEN // PHASES
NCHUNK = ROWS_PER_W * PHASES
NBUF = 8
DEPTH = 7
GROUPS = D // 16


def _body(x_hbm, tok_hbm, pos_hbm, out_hbm, idx_all, posv, *rest):
  bufs = rest[:NBUF]
  gsems = rest[NBUF:2 * NBUF]
  ssems = rest[2 * NBUF:]

  wid = lax.axis_index("s") * NC + lax.axis_index("c")
  chunk0 = wid * NCHUNK

  pltpu.sync_copy(x_hbm.at[pl.ds(chunk0, NCHUNK)], idx_all)
  pltpu.sync_copy(pos_hbm, posv)

  def start_gather(c, slot):
    pltpu.async_copy(tok_hbm.at[idx_all.at[c]], bufs[slot], gsems[slot])

  def wait_gather(c, slot):
    pltpu.make_async_copy(tok_hbm.at[idx_all.at[c]], bufs[slot],
                          gsems[slot]).wait()

  def start_store(c, slot):
    pltpu.async_copy(bufs[slot], out_hbm.at[chunk0 + c], ssems[slot])

  def wait_store(slot):
    pltpu.make_async_copy(bufs[slot], out_hbm.at[chunk0], ssems[slot]).wait()

  for c in range(DEPTH):
    start_gather(c, c)

  def chunk(c, slot):
    wait_gather(c, slot)
    buf = bufs[slot]
    h = lax.rem(c, PHASES)

    @plsc.parallel_loop(0, HALF, unroll=4)
    def _(r):
      for g in range(GROUPS):
        sl = pl.ds(g * 16, 16)
        buf[r, sl] = buf[r, sl] + posv[h, r, sl]

    start_store(c, slot)

    c2 = c + DEPTH
    s2_ = (slot + DEPTH) % NBUF

    @pl.when(c2 < NCHUNK)
    def _():
      @pl.when(c >= NBUF - DEPTH)
      def _():
        wait_store(s2_)
      start_gather(c2, s2_)

  @pl.loop(0, NCHUNK, step=NBUF)
  def _(k):
    for b in range(NBUF):
      chunk(k + b, b)

  for b in range(NBUF):
    wait_store(b)


@jax.jit
def kernel(x, token_table, pos_table):
  x2 = x.astype(jnp.int32).reshape(BATCH * PHASES, HALF)
  pos2 = pos_table.reshape(PHASES, HALF, D)
  mesh = plsc.VectorSubcoreMesh(core_axis_name="c", subcore_axis_name="s")
  fn = pl.kernel(
      _body,
      out_type=jax.ShapeDtypeStruct((BATCH * PHASES, HALF, D), jnp.float32),
      mesh=mesh,
      compiler_params=pltpu.CompilerParams(use_tc_tiling_on_sc=False),
      scratch_types=(
          [pltpu.VMEM((NCHUNK, HALF), jnp.int32),
           pltpu.VMEM((PHASES, HALF, D), jnp.float32)]
          + [pltpu.VMEM((HALF, D), jnp.float32)] * NBUF
          + [pltpu.SemaphoreType.DMA] * (2 * NBUF)
      ),
  )
  out = fn(x2, token_table, pos2)
  return out.reshape(BATCH, MAXLEN, D)


# speedup vs baseline: 1.0017x; 1.0017x over previous
import functools

import jax
import jax.numpy as jnp
from jax import lax
from jax.experimental import pallas as pl
from jax.experimental.pallas import tpu as pltpu
from jax.experimental.pallas import tpu_sc as plsc

MAXLEN = 200
VOCAB = 100000
D = 64
BATCH = 4096

NC = 2
NS = 16
NW = NC * NS
ROWS_PER_W = BATCH // NW
PHASES = 2
HALF = MAXLEN // PHASES
NCHUNK = ROWS_PER_W * PHASES
NBUF = 8
DEPTH = 6
GROUPS = D // 16


def _body(x_hbm, tok_hbm, pos_hbm, out_hbm, idx_all, posv, *rest):
  bufs = rest[:NBUF]
  gsems = rest[NBUF:2 * NBUF]
  ssems = rest[2 * NBUF:]

  wid = lax.axis_index("s") * NC + lax.axis_index("c")
  chunk0 = wid * NCHUNK

  pltpu.sync_copy(x_hbm.at[pl.ds(chunk0, NCHUNK)], idx_all)
  pltpu.sync_copy(pos_hbm, posv)

  def start_gather(c, slot):
    pltpu.async_copy(tok_hbm.at[idx_all.at[c]], bufs[slot], gsems[slot])

  def wait_gather(c, slot):
    pltpu.make_async_copy(tok_hbm.at[idx_all.at[c]], bufs[slot],
                          gsems[slot]).wait()

  def start_store(c, slot):
    pltpu.async_copy(bufs[slot], out_hbm.at[chunk0 + c], ssems[slot])

  def wait_store(slot):
    pltpu.make_async_copy(bufs[slot], out_hbm.at[chunk0], ssems[slot]).wait()

  for c in range(DEPTH):
    start_gather(c, c)

  def chunk(c, slot):
    wait_gather(c, slot)
    buf = bufs[slot]
    h = lax.rem(c, PHASES)

    @plsc.parallel_loop(0, HALF, unroll=4)
    def _(r):
      for g in range(GROUPS):
        sl = pl.ds(g * 16, 16)
        buf[r, sl] = buf[r, sl] + posv[h, r, sl]

    start_store(c, slot)

    c2 = c + DEPTH
    s2_ = (slot + DEPTH) % NBUF

    @pl.when(c2 < NCHUNK)
    def _():
      @pl.when(c >= NBUF - DEPTH)
      def _():
        wait_store(s2_)
      start_gather(c2, s2_)

  @pl.loop(0, NCHUNK, step=NBUF)
  def _(k):
    for b in range(NBUF):
      chunk(k + b, b)

  for b in range(NBUF):
    wait_store(b)


@jax.jit
def kernel(x, token_table, pos_table):
  x2 = x.astype(jnp.int32).reshape(BATCH * PHASES, HALF)
  pos2 = pos_table.reshape(PHASES, HALF, D)
  mesh = plsc.VectorSubcoreMesh(core_axis_name="c", subcore_axis_name="s")
  fn = pl.kernel(
      _body,
      out_type=jax.ShapeDtypeStruct((BATCH * PHASES, HALF, D), jnp.float32),
      mesh=mesh,
      compiler_params=pltpu.CompilerParams(use_tc_tiling_on_sc=False),
      scratch_types=(
          [pltpu.VMEM((NCHUNK, HALF), jnp.int32),
           pltpu.VMEM((PHASES, HALF, D), jnp.float32)]
          + [pltpu.VMEM((HALF, D), jnp.float32)] * NBUF
          + [pltpu.SemaphoreType.DMA] * (2 * NBUF)
      ),
  )
  out = fn(x2, token_table, pos2)
  return out.reshape(BATCH, MAXLEN, D)
